# Initial kernel scaffold; baseline (speedup 1.0000x reference)
import jax
import jax.numpy as jnp
from jax.experimental import pallas as pl


def kernel(x, edge_index, W1, b1, W2, b2, W3, b3):
    raise NotImplementedError("write your pallas kernel here")


# speedup vs baseline: 6.5541x; 6.5541x over previous
import functools

import jax
import jax.numpy as jnp
from jax import lax
from jax.experimental import pallas as pl
from jax.experimental.pallas import tpu as pltpu
from jax.experimental.pallas import tpu_sc as plsc

N = 10000
E = 160000
NPAD = 10240
NS = 16
NC = 2
ZR = NPAD // NS
K = 80
EPT = E // NS
NCH = EPT // K
EPT3 = E // (NS * NC)
K3 = 40
NCH3 = EPT3 // K3

_mesh = lambda: plsc.VectorSubcoreMesh(core_axis_name="c", subcore_axis_name="s")


def _make_deg():
    @functools.partial(
        pl.kernel,
        out_type=[jax.ShapeDtypeStruct((NPAD, 128), jnp.float32),
                  jax.ShapeDtypeStruct((NPAD, 128), jnp.float32)],
        mesh=_mesh(),
        scratch_types=[
            pltpu.VMEM_SHARED((NPAD, 128), jnp.float32),
            pltpu.VMEM((K3, 128), jnp.float32),
            pltpu.VMEM((1, K3), jnp.int32),
        ],
    )
    def deg_kernel(dst_hbm, z_hbm, ones_hbm, d0_hbm, d1_hbm,
                   deg_sh, ones_v, idx_v):
        cid = lax.axis_index("c")
        sid = lax.axis_index("s")
        pltpu.sync_copy(z_hbm.at[pl.ds(sid * ZR, ZR)],
                        deg_sh.at[pl.ds(sid * ZR, ZR)])
        pltpu.sync_copy(ones_hbm, ones_v)
        plsc.subcore_barrier()

        base = (cid * NS + sid) * EPT3

        def chunk(i, carry):
            off = base + i * K3
            pltpu.sync_copy(dst_hbm.at[pl.ds(off, K3)], idx_v.at[0])
            pltpu.sync_copy(ones_v, deg_sh.at[idx_v.at[0]], add=True)
            return carry

        lax.fori_loop(0, NCH3, chunk, 0)
        plsc.subcore_barrier()

        @pl.when(cid == 0)
        def _():
            pltpu.sync_copy(deg_sh.at[pl.ds(sid * ZR, ZR)],
                            d0_hbm.at[pl.ds(sid * ZR, ZR)])

        @pl.when(cid == 1)
        def _():
            pltpu.sync_copy(deg_sh.at[pl.ds(sid * ZR, ZR)],
                            d1_hbm.at[pl.ds(sid * ZR, ZR)])

    return deg_kernel


def _make_agg():
    @functools.partial(
        pl.kernel,
        out_type=[jax.ShapeDtypeStruct((NPAD, 128), jnp.float32),
                  jax.ShapeDtypeStruct((NPAD, 128), jnp.float32)],
        mesh=_mesh(),
        scratch_types=[
            pltpu.VMEM_SHARED((NPAD, 128), jnp.float32),
            pltpu.VMEM((1, K), jnp.int32),
            pltpu.VMEM((1, K), jnp.int32),
            pltpu.VMEM((1, K, 128), jnp.float32),
            pltpu.SemaphoreType.DMA,
        ],
    )
    def agg_kernel(hl_hbm, hr_hbm, src_hbm, dst_hbm, z_hbm,
                   accl_hbm, accr_hbm, acc_sh, sbuf, dbuf, rbuf, sem):
        cid = lax.axis_index("c")
        sid = lax.axis_index("s")
        pltpu.sync_copy(z_hbm.at[pl.ds(sid * ZR, ZR)],
                        acc_sh.at[pl.ds(sid * ZR, ZR)])
        plsc.subcore_barrier()
        base = sid * EPT

        def run(h_hbm, out_hbm):
            def chunk(i, carry):
                off = base + i * K
                pltpu.sync_copy(src_hbm.at[pl.ds(off, K)], sbuf.at[0])
                pltpu.sync_copy(dst_hbm.at[pl.ds(off, K)], dbuf.at[0])
                pltpu.async_copy(h_hbm.at[sbuf.at[0]], rbuf.at[0], sem).wait()
                pltpu.sync_copy(rbuf.at[0], acc_sh.at[dbuf.at[0]], add=True)
                return carry

            lax.fori_loop(0, NCH, chunk, 0)
            plsc.subcore_barrier()
            pltpu.sync_copy(acc_sh.at[pl.ds(sid * ZR, ZR)],
                            out_hbm.at[pl.ds(sid * ZR, ZR)])

        @pl.when(cid == 0)
        def _():
            run(hl_hbm, accl_hbm)

        @pl.when(cid == 1)
        def _():
            run(hr_hbm, accr_hbm)

    return agg_kernel


def _make_agg_l3():
    @functools.partial(
        pl.kernel,
        out_type=[jax.ShapeDtypeStruct((NPAD, 128), jnp.float32),
                  jax.ShapeDtypeStruct((NPAD, 128), jnp.float32)],
        mesh=_mesh(),
        scratch_types=[
            pltpu.VMEM_SHARED((NPAD, 128), jnp.float32),
            pltpu.VMEM((1, K3), jnp.int32),
            pltpu.VMEM((1, K3), jnp.int32),
            pltpu.VMEM((1, K3, 128), jnp.float32),
            pltpu.SemaphoreType.DMA,
        ],
    )
    def agg3_kernel(h_hbm, src_hbm, dst_hbm, z_hbm,
                    acc0_hbm, acc1_hbm, acc_sh, sbuf, dbuf, rbuf, sem):
        cid = lax.axis_index("c")
        sid = lax.axis_index("s")
        pltpu.sync_copy(z_hbm.at[pl.ds(sid * ZR, ZR)],
                        acc_sh.at[pl.ds(sid * ZR, ZR)])
        plsc.subcore_barrier()
        base = (cid * NS + sid) * EPT3

        def chunk(i, carry):
            off = base + i * K3
            pltpu.sync_copy(src_hbm.at[pl.ds(off, K3)], sbuf.at[0])
            pltpu.sync_copy(dst_hbm.at[pl.ds(off, K3)], dbuf.at[0])
            pltpu.async_copy(h_hbm.at[sbuf.at[0]], rbuf.at[0], sem).wait()
            pltpu.sync_copy(rbuf.at[0], acc_sh.at[dbuf.at[0]], add=True)
            return carry

        lax.fori_loop(0, NCH3, chunk, 0)
        plsc.subcore_barrier()

        @pl.when(cid == 0)
        def _():
            pltpu.sync_copy(acc_sh.at[pl.ds(sid * ZR, ZR)],
                            acc0_hbm.at[pl.ds(sid * ZR, ZR)])

        @pl.when(cid == 1)
        def _():
            pltpu.sync_copy(acc_sh.at[pl.ds(sid * ZR, ZR)],
                            acc1_hbm.at[pl.ds(sid * ZR, ZR)])

    return agg3_kernel


_deg = _make_deg()
_agg = _make_agg()
_agg3 = _make_agg_l3()


BM = 512
GRID = NPAD // BM


def _dinv(d0, d1):
    return lax.rsqrt(d0[:, :1] + d1[:, :1] + 1.0)


def _b1_body(x_ref, w_ref, d0_ref, d1_ref, ol_ref, or_ref):
    di = _dinv(d0_ref[...], d1_ref[...])
    h = jnp.dot(x_ref[...], w_ref[...], preferred_element_type=jnp.float32)
    ht = h * di
    ol_ref[...] = ht[:, :128]
    or_ref[...] = ht[:, 128:]


_b1 = pl.pallas_call(
    _b1_body,
    grid=(GRID,),
    in_specs=[
        pl.BlockSpec((BM, 256), lambda i: (i, 0)),
        pl.BlockSpec((256, 256), lambda i: (0, 0)),
        pl.BlockSpec((BM, 128), lambda i: (i, 0)),
        pl.BlockSpec((BM, 128), lambda i: (i, 0)),
    ],
    out_specs=[
        pl.BlockSpec((BM, 128), lambda i: (i, 0)),
        pl.BlockSpec((BM, 128), lambda i: (i, 0)),
    ],
    out_shape=[jax.ShapeDtypeStruct((NPAD, 128), jnp.float32)] * 2,
)


def _b2_body(al_ref, ar_ref, hl_ref, hr_ref, d0_ref, d1_ref, b_ref, w_ref,
             ol_ref, or_ref):
    di = _dinv(d0_ref[...], d1_ref[...])
    s = jnp.concatenate(
        [al_ref[...] + hl_ref[...], ar_ref[...] + hr_ref[...]], axis=1)
    z = jnp.maximum(s * di + b_ref[0:1, :], 0.0)
    o = jnp.dot(z, w_ref[...], preferred_element_type=jnp.float32) * di
    ol_ref[...] = o[:, :128]
    or_ref[...] = o[:, 128:]


_b2 = pl.pallas_call(
    _b2_body,
    grid=(GRID,),
    in_specs=[
        pl.BlockSpec((BM, 128), lambda i: (i, 0)),
        pl.BlockSpec((BM, 128), lambda i: (i, 0)),
        pl.BlockSpec((BM, 128), lambda i: (i, 0)),
        pl.BlockSpec((BM, 128), lambda i: (i, 0)),
        pl.BlockSpec((BM, 128), lambda i: (i, 0)),
        pl.BlockSpec((BM, 128), lambda i: (i, 0)),
        pl.BlockSpec((8, 256), lambda i: (0, 0)),
        pl.BlockSpec((256, 256), lambda i: (0, 0)),
    ],
    out_specs=[
        pl.BlockSpec((BM, 128), lambda i: (i, 0)),
        pl.BlockSpec((BM, 128), lambda i: (i, 0)),
    ],
    out_shape=[jax.ShapeDtypeStruct((NPAD, 128), jnp.float32)] * 2,
)


def _b3_body(al_ref, ar_ref, hl_ref, hr_ref, d0_ref, d1_ref, b_ref, w_ref,
             o_ref):
    di = _dinv(d0_ref[...], d1_ref[...])
    s = jnp.concatenate(
        [al_ref[...] + hl_ref[...], ar_ref[...] + hr_ref[...]], axis=1)
    z = jnp.maximum(s * di + b_ref[0:1, :], 0.0)
    o = jnp.dot(z, w_ref[...], preferred_element_type=jnp.float32) * di
    o_ref[...] = jnp.concatenate([o, jnp.zeros((BM, 64), jnp.float32)], axis=1)


_b3 = pl.pallas_call(
    _b3_body,
    grid=(GRID,),
    in_specs=[
        pl.BlockSpec((BM, 128), lambda i: (i, 0)),
        pl.BlockSpec((BM, 128), lambda i: (i, 0)),
        pl.BlockSpec((BM, 128), lambda i: (i, 0)),
        pl.BlockSpec((BM, 128), lambda i: (i, 0)),
        pl.BlockSpec((BM, 128), lambda i: (i, 0)),
        pl.BlockSpec((BM, 128), lambda i: (i, 0)),
        pl.BlockSpec((8, 256), lambda i: (0, 0)),
        pl.BlockSpec((256, 64), lambda i: (0, 0)),
    ],
    out_specs=pl.BlockSpec((BM, 128), lambda i: (i, 0)),
    out_shape=jax.ShapeDtypeStruct((NPAD, 128), jnp.float32),
)


def _b4_body(a0_ref, a1_ref, h_ref, d0_ref, d1_ref, b_ref, o_ref):
    di = _dinv(d0_ref[...], d1_ref[...])
    s = a0_ref[...] + a1_ref[...] + h_ref[...]
    o_ref[...] = s[:, :64] * di + b_ref[0:1, :]


_b4 = pl.pallas_call(
    _b4_body,
    grid=(GRID,),
    in_specs=[
        pl.BlockSpec((BM, 128), lambda i: (i, 0)),
        pl.BlockSpec((BM, 128), lambda i: (i, 0)),
        pl.BlockSpec((BM, 128), lambda i: (i, 0)),
        pl.BlockSpec((BM, 128), lambda i: (i, 0)),
        pl.BlockSpec((BM, 128), lambda i: (i, 0)),
        pl.BlockSpec((8, 64), lambda i: (0, 0)),
    ],
    out_specs=pl.BlockSpec((BM, 64), lambda i: (i, 0)),
    out_shape=jax.ShapeDtypeStruct((NPAD, 64), jnp.float32),
)


def kernel(x, edge_index, W1, b1, W2, b2, W3, b3):
    src = edge_index[0].astype(jnp.int32)
    dst = edge_index[1].astype(jnp.int32)
    xp = jnp.pad(x, ((0, NPAD - N), (0, 0)))
    z128 = jnp.zeros((NPAD, 128), jnp.float32)
    ones = jnp.ones((K3, 128), jnp.float32)
    b1t = jnp.tile(b1[None, :], (8, 1))
    b2t = jnp.tile(b2[None, :], (8, 1))
    b3t = jnp.tile(b3[None, :], (8, 1))

    d0, d1 = _deg(dst, z128, ones)
    h1l, h1r = _b1(xp, W1, d0, d1)
    a1l, a1r = _agg(h1l, h1r, src, dst, z128)
    h2l, h2r = _b2(a1l, a1r, h1l, h1r, d0, d1, b1t, W2)
    a2l, a2r = _agg(h2l, h2r, src, dst, z128)
    h3 = _b3(a2l, a2r, h2l, h2r, d0, d1, b2t, W3)
    a30, a31 = _agg3(h3, src, dst, z128)
    out = _b4(a30, a31, h3, d0, d1, b3t)
    return out[:N]


# speedup vs baseline: 14.0319x; 2.1409x over previous
import functools

import jax
import jax.numpy as jnp
from jax import lax
from jax.experimental import pallas as pl
from jax.experimental.pallas import tpu as pltpu
from jax.experimental.pallas import tpu_sc as plsc

N = 10000
E = 160000
NPAD = 10240
NS = 16
NC = 2
ZR = NPAD // NS
K = 40
EPT = E // NS
NCH = EPT // K
EPT3 = E // (NS * NC)
K3 = 40
NCH3 = EPT3 // K3

_mesh = lambda: plsc.VectorSubcoreMesh(core_axis_name="c", subcore_axis_name="s")


def _make_deg():
    @functools.partial(
        pl.kernel,
        out_type=[jax.ShapeDtypeStruct((NPAD, 128), jnp.float32),
                  jax.ShapeDtypeStruct((NPAD, 128), jnp.float32)],
        mesh=_mesh(),
        scratch_types=[
            pltpu.VMEM_SHARED((NPAD, 128), jnp.float32),
            pltpu.VMEM((K3, 128), jnp.float32),
            pltpu.VMEM((1, K3), jnp.int32),
        ],
    )
    def deg_kernel(dst_hbm, z_hbm, ones_hbm, d0_hbm, d1_hbm,
                   deg_sh, ones_v, idx_v):
        cid = lax.axis_index("c")
        sid = lax.axis_index("s")
        pltpu.sync_copy(z_hbm.at[pl.ds(sid * ZR, ZR)],
                        deg_sh.at[pl.ds(sid * ZR, ZR)])
        pltpu.sync_copy(ones_hbm, ones_v)
        plsc.subcore_barrier()

        base = (cid * NS + sid) * EPT3

        def chunk(i, carry):
            off = base + i * K3
            pltpu.sync_copy(dst_hbm.at[pl.ds(off, K3)], idx_v.at[0])
            pltpu.sync_copy(ones_v, deg_sh.at[idx_v.at[0]], add=True)
            return carry

        lax.fori_loop(0, NCH3, chunk, 0)
        plsc.subcore_barrier()

        @pl.when(cid == 0)
        def _():
            pltpu.sync_copy(deg_sh.at[pl.ds(sid * ZR, ZR)],
                            d0_hbm.at[pl.ds(sid * ZR, ZR)])

        @pl.when(cid == 1)
        def _():
            pltpu.sync_copy(deg_sh.at[pl.ds(sid * ZR, ZR)],
                            d1_hbm.at[pl.ds(sid * ZR, ZR)])

    return deg_kernel


NBUF = 5
NOUT = NCH // NBUF
NOUT3 = NCH3 // NBUF


def _pipe(h_hbm, dst_hbm, acc_sh, sbuf, dibuf, rbuf, sems, ebase, k, nout):
    gsems, isems, ssems = sems[:NBUF], sems[NBUF:2 * NBUF], sems[2 * NBUF:]

    def g_start(i, b):
        pltpu.async_copy(h_hbm.at[sbuf.at[pl.ds(i * k, k)]], rbuf.at[b],
                         gsems[b])
        pltpu.async_copy(dst_hbm.at[pl.ds(ebase + i * k, k)], dibuf.at[b],
                         isems[b])

    def g_wait(b):
        pltpu.make_async_copy(h_hbm.at[sbuf.at[pl.ds(0, k)]], rbuf.at[b],
                              gsems[b]).wait()
        pltpu.make_async_copy(dst_hbm.at[pl.ds(ebase, k)], dibuf.at[b],
                              isems[b]).wait()

    def s_start(b):
        pltpu.async_copy(rbuf.at[b], acc_sh.at[dibuf.at[b]], ssems[b],
                         add=True)

    def s_wait(b):
        pltpu.make_async_copy(rbuf.at[b], acc_sh.at[dibuf.at[0]],
                              ssems[b]).wait()

    for b in range(NBUF):
        g_start(b, b)

    def block(t, carry):
        for b in range(NBUF):
            g_wait(b)
            s_start(b)

        @pl.when(t < nout - 1)
        def _():
            for b in range(NBUF):
                s_wait(b)
                g_start((t + 1) * NBUF + b, b)

        return carry

    lax.fori_loop(0, nout, block, 0)
    for b in range(NBUF):
        s_wait(b)


def _sc_scratch(ept, k):
    return [
        pltpu.VMEM_SHARED((NPAD, 128), jnp.float32),
        pltpu.VMEM((ept,), jnp.int32),
        pltpu.VMEM((NBUF, k), jnp.int32),
        pltpu.VMEM((NBUF, k, 128), jnp.float32),
    ] + [pltpu.SemaphoreType.DMA] * (3 * NBUF)


def _make_agg():
    @functools.partial(
        pl.kernel,
        out_type=[jax.ShapeDtypeStruct((NPAD, 128), jnp.float32),
                  jax.ShapeDtypeStruct((NPAD, 128), jnp.float32)],
        mesh=_mesh(),
        scratch_types=_sc_scratch(EPT, K),
    )
    def agg_kernel(hl_hbm, hr_hbm, src_hbm, dst_hbm, z_hbm,
                   accl_hbm, accr_hbm, acc_sh, sbuf, dibuf, rbuf, *sems):
        cid = lax.axis_index("c")
        sid = lax.axis_index("s")
        pltpu.sync_copy(src_hbm.at[pl.ds(sid * EPT, EPT)], sbuf)
        pltpu.sync_copy(z_hbm.at[pl.ds(sid * ZR, ZR)],
                        acc_sh.at[pl.ds(sid * ZR, ZR)])
        plsc.subcore_barrier()

        def run(h_hbm, out_hbm):
            _pipe(h_hbm, dst_hbm, acc_sh, sbuf, dibuf, rbuf,
                  sems, sid * EPT, K, NOUT)
            plsc.subcore_barrier()
            pltpu.sync_copy(acc_sh.at[pl.ds(sid * ZR, ZR)],
                            out_hbm.at[pl.ds(sid * ZR, ZR)])

        @pl.when(cid == 0)
        def _():
            run(hl_hbm, accl_hbm)

        @pl.when(cid == 1)
        def _():
            run(hr_hbm, accr_hbm)

    return agg_kernel


def _make_agg_l3():
    @functools.partial(
        pl.kernel,
        out_type=[jax.ShapeDtypeStruct((NPAD, 128), jnp.float32),
                  jax.ShapeDtypeStruct((NPAD, 128), jnp.float32)],
        mesh=_mesh(),
        scratch_types=_sc_scratch(EPT3, K3),
    )
    def agg3_kernel(h_hbm, src_hbm, dst_hbm, z_hbm,
                    acc0_hbm, acc1_hbm, acc_sh, sbuf, dibuf, rbuf, *sems):
        cid = lax.axis_index("c")
        sid = lax.axis_index("s")
        w = cid * NS + sid
        pltpu.sync_copy(src_hbm.at[pl.ds(w * EPT3, EPT3)], sbuf)
        pltpu.sync_copy(z_hbm.at[pl.ds(sid * ZR, ZR)],
                        acc_sh.at[pl.ds(sid * ZR, ZR)])
        plsc.subcore_barrier()
        _pipe(h_hbm, dst_hbm, acc_sh, sbuf, dibuf, rbuf,
              sems, w * EPT3, K3, NOUT3)
        plsc.subcore_barrier()

        @pl.when(cid == 0)
        def _():
            pltpu.sync_copy(acc_sh.at[pl.ds(sid * ZR, ZR)],
                            acc0_hbm.at[pl.ds(sid * ZR, ZR)])

        @pl.when(cid == 1)
        def _():
            pltpu.sync_copy(acc_sh.at[pl.ds(sid * ZR, ZR)],
                            acc1_hbm.at[pl.ds(sid * ZR, ZR)])

    return agg3_kernel


_deg = _make_deg()
_agg = _make_agg()
_agg3 = _make_agg_l3()


BM = 512
GRID = NPAD // BM


def _dinv(d0, d1):
    return lax.rsqrt(d0[:, :1] + d1[:, :1] + 1.0)


def _b1_body(x_ref, w_ref, d0_ref, d1_ref, ol_ref, or_ref):
    di = _dinv(d0_ref[...], d1_ref[...])
    h = jnp.dot(x_ref[...], w_ref[...], preferred_element_type=jnp.float32)
    ht = h * di
    ol_ref[...] = ht[:, :128]
    or_ref[...] = ht[:, 128:]


_b1 = pl.pallas_call(
    _b1_body,
    grid=(GRID,),
    in_specs=[
        pl.BlockSpec((BM, 256), lambda i: (i, 0)),
        pl.BlockSpec((256, 256), lambda i: (0, 0)),
        pl.BlockSpec((BM, 128), lambda i: (i, 0)),
        pl.BlockSpec((BM, 128), lambda i: (i, 0)),
    ],
    out_specs=[
        pl.BlockSpec((BM, 128), lambda i: (i, 0)),
        pl.BlockSpec((BM, 128), lambda i: (i, 0)),
    ],
    out_shape=[jax.ShapeDtypeStruct((NPAD, 128), jnp.float32)] * 2,
)


def _b2_body(al_ref, ar_ref, hl_ref, hr_ref, d0_ref, d1_ref, b_ref, w_ref,
             ol_ref, or_ref):
    di = _dinv(d0_ref[...], d1_ref[...])
    s = jnp.concatenate(
        [al_ref[...] + hl_ref[...], ar_ref[...] + hr_ref[...]], axis=1)
    z = jnp.maximum(s * di + b_ref[0:1, :], 0.0)
    o = jnp.dot(z, w_ref[...], preferred_element_type=jnp.float32) * di
    ol_ref[...] = o[:, :128]
    or_ref[...] = o[:, 128:]


_b2 = pl.pallas_call(
    _b2_body,
    grid=(GRID,),
    in_specs=[
        pl.BlockSpec((BM, 128), lambda i: (i, 0)),
        pl.BlockSpec((BM, 128), lambda i: (i, 0)),
        pl.BlockSpec((BM, 128), lambda i: (i, 0)),
        pl.BlockSpec((BM, 128), lambda i: (i, 0)),
        pl.BlockSpec((BM, 128), lambda i: (i, 0)),
        pl.BlockSpec((BM, 128), lambda i: (i, 0)),
        pl.BlockSpec((8, 256), lambda i: (0, 0)),
        pl.BlockSpec((256, 256), lambda i: (0, 0)),
    ],
    out_specs=[
        pl.BlockSpec((BM, 128), lambda i: (i, 0)),
        pl.BlockSpec((BM, 128), lambda i: (i, 0)),
    ],
    out_shape=[jax.ShapeDtypeStruct((NPAD, 128), jnp.float32)] * 2,
)


def _b3_body(al_ref, ar_ref, hl_ref, hr_ref, d0_ref, d1_ref, b_ref, w_ref,
             o_ref):
    di = _dinv(d0_ref[...], d1_ref[...])
    s = jnp.concatenate(
        [al_ref[...] + hl_ref[...], ar_ref[...] + hr_ref[...]], axis=1)
    z = jnp.maximum(s * di + b_ref[0:1, :], 0.0)
    o = jnp.dot(z, w_ref[...], preferred_element_type=jnp.float32) * di
    o_ref[...] = jnp.concatenate([o, jnp.zeros((BM, 64), jnp.float32)], axis=1)


_b3 = pl.pallas_call(
    _b3_body,
    grid=(GRID,),
    in_specs=[
        pl.BlockSpec((BM, 128), lambda i: (i, 0)),
        pl.BlockSpec((BM, 128), lambda i: (i, 0)),
        pl.BlockSpec((BM, 128), lambda i: (i, 0)),
        pl.BlockSpec((BM, 128), lambda i: (i, 0)),
        pl.BlockSpec((BM, 128), lambda i: (i, 0)),
        pl.BlockSpec((BM, 128), lambda i: (i, 0)),
        pl.BlockSpec((8, 256), lambda i: (0, 0)),
        pl.BlockSpec((256, 64), lambda i: (0, 0)),
    ],
    out_specs=pl.BlockSpec((BM, 128), lambda i: (i, 0)),
    out_shape=jax.ShapeDtypeStruct((NPAD, 128), jnp.float32),
)


def _b4_body(a0_ref, a1_ref, h_ref, d0_ref, d1_ref, b_ref, o_ref):
    di = _dinv(d0_ref[...], d1_ref[...])
    s = a0_ref[...] + a1_ref[...] + h_ref[...]
    o_ref[...] = s[:, :64] * di + b_ref[0:1, :]


_b4 = pl.pallas_call(
    _b4_body,
    grid=(GRID,),
    in_specs=[
        pl.BlockSpec((BM, 128), lambda i: (i, 0)),
        pl.BlockSpec((BM, 128), lambda i: (i, 0)),
        pl.BlockSpec((BM, 128), lambda i: (i, 0)),
        pl.BlockSpec((BM, 128), lambda i: (i, 0)),
        pl.BlockSpec((BM, 128), lambda i: (i, 0)),
        pl.BlockSpec((8, 64), lambda i: (0, 0)),
    ],
    out_specs=pl.BlockSpec((BM, 64), lambda i: (i, 0)),
    out_shape=jax.ShapeDtypeStruct((NPAD, 64), jnp.float32),
)


def kernel(x, edge_index, W1, b1, W2, b2, W3, b3):
    src = edge_index[0].astype(jnp.int32)
    dst = edge_index[1].astype(jnp.int32)
    xp = jnp.pad(x, ((0, NPAD - N), (0, 0)))
    z128 = jnp.zeros((NPAD, 128), jnp.float32)
    ones = jnp.ones((K3, 128), jnp.float32)
    b1t = jnp.tile(b1[None, :], (8, 1))
    b2t = jnp.tile(b2[None, :], (8, 1))
    b3t = jnp.tile(b3[None, :], (8, 1))

    d0, d1 = _deg(dst, z128, ones)
    h1l, h1r = _b1(xp, W1, d0, d1)
    a1l, a1r = _agg(h1l, h1r, src, dst, z128)
    h2l, h2r = _b2(a1l, a1r, h1l, h1r, d0, d1, b1t, W2)
    a2l, a2r = _agg(h2l, h2r, src, dst, z128)
    h3 = _b3(a2l, a2r, h2l, h2r, d0, d1, b2t, W3)
    a30, a31 = _agg3(h3, src, dst, z128)
    out = _b4(a30, a31, h3, d0, d1, b3t)
    return out[:N]
